# baseline (speedup 1.0000x reference)
import functools

import jax
import jax.numpy as jnp
from jax import lax
from jax.experimental import pallas as pl
from jax.experimental.pallas import tpu as pltpu
from jax.experimental.pallas import tpu_sc as plsc

NN = 10000
EE = 320000
DIN = 128
HH = 64
BB = 256

NC = 2
NS = 16
NW = NC * NS

CHUNK = 112
NCH = 90
EPW = CHUNK * NCH
EPAD = EPW * NW
NACC = 10112
STRIPE = NACC // NS

NPAD = 10240
RPT = NPAD // NW
NG = RPT // 16
BCH = 64
NBCH = RPT // BCH
PB = 384
PSTR = PB // NS

_MESH = plsc.VectorSubcoreMesh(core_axis_name="c", subcore_axis_name="s")


def _zero16():
    return jnp.zeros((16,), jnp.float32)


def _dyn_gather(x, idx):
    return lax.gather(
        x, idx[:, None],
        lax.GatherDimensionNumbers(offset_dims=(), collapsed_slice_dims=(0,),
                                   start_index_map=(0,)),
        (1,), mode=lax.GatherScatterMode.PROMISE_IN_BOUNDS)


@functools.partial(
    pl.kernel,
    out_type=jax.ShapeDtypeStruct((NC * NACC,), jnp.float32),
    mesh=_MESH,
    scratch_types=[
        pltpu.VMEM((NCH, CHUNK), jnp.int32),
        pltpu.VMEM((CHUNK,), jnp.float32),
        pltpu.VMEM((CHUNK,), jnp.float32),
        pltpu.VMEM((STRIPE,), jnp.float32),
        pltpu.VMEM_SHARED((NACC,), jnp.float32),
    ],
)
def _deg_kernel(dsts, out, dst_v, ones_v, zbuf, stage_v, acc):
    c = lax.axis_index("c")
    s = lax.axis_index("s")
    w = c * NS + s
    one16 = jnp.ones((16,), jnp.float32)
    for k in range(CHUNK // 16):
        ones_v[pl.ds(16 * k, 16)] = one16
        zbuf[pl.ds(16 * k, 16)] = _zero16()
    base = s * STRIPE
    nfull = STRIPE // CHUNK
    tail = STRIPE - nfull * CHUNK
    for q in range(nfull):
        pltpu.sync_copy(zbuf, acc.at[pl.ds(base + q * CHUNK, CHUNK)])
    pltpu.sync_copy(zbuf.at[pl.ds(0, tail)],
                    acc.at[pl.ds(base + nfull * CHUNK, tail)])
    pltpu.sync_copy(dsts.at[w], dst_v)
    plsc.subcore_barrier()

    def step(j, carry):
        pltpu.sync_copy(ones_v, acc.at[dst_v.at[j]], add=True)
        return carry

    lax.fori_loop(0, NCH, step, 0)
    plsc.subcore_barrier()
    pltpu.sync_copy(acc.at[pl.ds(base, STRIPE)], stage_v)
    pltpu.sync_copy(stage_v, out.at[pl.ds(c * NACC + base, STRIPE)])


@functools.partial(
    pl.kernel,
    out_type=jax.ShapeDtypeStruct((NC * NACC, HH), jnp.float32),
    mesh=_MESH,
    compiler_params=pltpu.CompilerParams(use_tc_tiling_on_sc=False),
    scratch_types=[
        pltpu.VMEM((NCH, CHUNK), jnp.int32),
        pltpu.VMEM((NCH, CHUNK), jnp.int32),
        pltpu.VMEM((CHUNK, HH), jnp.float32),
        pltpu.VMEM((CHUNK, HH), jnp.float32),
        pltpu.VMEM_SHARED((NACC, HH), jnp.float32),
        pltpu.SemaphoreType.DMA,
        pltpu.SemaphoreType.DMA,
        pltpu.SemaphoreType.DMA,
        pltpu.SemaphoreType.DMA,
    ],
)
def _edge_kernel(t, srcs, dsts, out, src_v, dst_v, rows_a, rows_b, acc,
                 sem_a, sem_b, sem_sa, sem_sb):
    c = lax.axis_index("c")
    s = lax.axis_index("s")
    w = c * NS + s
    base = s * STRIPE

    def zb(i, carry):
        for k in range(HH // 16):
            rows_a[i, pl.ds(16 * k, 16)] = _zero16()
        return carry

    lax.fori_loop(0, CHUNK, zb, 0)
    nfull = STRIPE // CHUNK
    tail = STRIPE - nfull * CHUNK
    for q in range(nfull):
        pltpu.sync_copy(rows_a, acc.at[pl.ds(base + q * CHUNK, CHUNK)])
    pltpu.sync_copy(rows_a.at[pl.ds(0, tail)],
                    acc.at[pl.ds(base + nfull * CHUNK, tail)])
    pltpu.sync_copy(srcs.at[w], src_v)
    pltpu.sync_copy(dsts.at[w], dst_v)
    plsc.subcore_barrier()

    pltpu.async_copy(t.at[src_v.at[0]], rows_a, sem_a)
    pltpu.async_copy(t.at[src_v.at[1]], rows_b, sem_b)

    def step(g, carry):
        j0 = 2 * g
        pltpu.make_async_copy(t.at[src_v.at[j0]], rows_a, sem_a).wait()
        pltpu.async_copy(rows_a, acc.at[dst_v.at[j0]], sem_sa, add=True)
        pltpu.make_async_copy(t.at[src_v.at[j0 + 1]], rows_b, sem_b).wait()
        pltpu.async_copy(rows_b, acc.at[dst_v.at[j0 + 1]], sem_sb, add=True)

        @pl.when(j0 + 2 < NCH)
        def _():
            pltpu.make_async_copy(rows_a, acc.at[dst_v.at[j0]],
                                  sem_sa).wait()
            pltpu.async_copy(t.at[src_v.at[j0 + 2]], rows_a, sem_a)

        @pl.when(j0 + 3 < NCH)
        def _():
            pltpu.make_async_copy(rows_b, acc.at[dst_v.at[j0 + 1]],
                                  sem_sb).wait()
            pltpu.async_copy(t.at[src_v.at[j0 + 3]], rows_b, sem_b)

        return carry

    lax.fori_loop(0, NCH // 2, step, 0)
    pltpu.make_async_copy(rows_a, acc.at[dst_v.at[0]], sem_sa).wait()
    pltpu.make_async_copy(rows_b, acc.at[dst_v.at[1]], sem_sb).wait()
    plsc.subcore_barrier()
    for q in range(nfull):
        pltpu.sync_copy(acc.at[pl.ds(base + q * CHUNK, CHUNK)], rows_a)
        pltpu.sync_copy(rows_a,
                        out.at[pl.ds(c * NACC + base + q * CHUNK, CHUNK)])
    pltpu.sync_copy(acc.at[pl.ds(base + nfull * CHUNK, tail)],
                    rows_a.at[pl.ds(0, tail)])
    pltpu.sync_copy(rows_a.at[pl.ds(0, tail)],
                    out.at[pl.ds(c * NACC + base + nfull * CHUNK, tail)])


@functools.partial(
    pl.kernel,
    out_type=(
        jax.ShapeDtypeStruct((NC * BB, HH), jnp.float32),
        jax.ShapeDtypeStruct((NC * BB,), jnp.float32),
        jax.ShapeDtypeStruct((NW * BB * HH,), jnp.float32),
    ),
    mesh=_MESH,
    compiler_params=pltpu.CompilerParams(needs_layout_passes=False,
                                         use_tc_tiling_on_sc=False),
    scratch_types=[
        pltpu.VMEM((RPT, HH), jnp.float32),
        pltpu.VMEM((RPT * HH,), jnp.float32),
        pltpu.VMEM((NBCH, BCH), jnp.int32),
        pltpu.VMEM((BCH,), jnp.float32),
        pltpu.VMEM((PSTR, HH), jnp.float32),
        pltpu.VMEM((PSTR,), jnp.float32),
        pltpu.VMEM((PB * HH,), jnp.float32),
        pltpu.VMEM((16, HH), jnp.float32),
        pltpu.VMEM((16,), jnp.float32),
        pltpu.VMEM_SHARED((PB, HH), jnp.float32),
        pltpu.VMEM_SHARED((PB,), jnp.float32),
    ],
)
def _pool_kernel(h2d, hflat, bat2, out_sum, out_cnt, out_max,
                 h2d_v, hflat_v, bat2_v, ones_v, zrows, zrow1,
                 maxbuf, sstage, cstage, sum_acc, cnt_acc):
    c = lax.axis_index("c")
    s = lax.axis_index("s")
    w = c * NS + s
    one16 = jnp.ones((16,), jnp.float32)
    iota = lax.iota(jnp.int32, 16)
    ninf16 = jnp.full((16,), -jnp.inf, jnp.float32)

    for k in range(BCH // 16):
        ones_v[pl.ds(16 * k, 16)] = one16
    for i in range(PSTR):
        for k in range(HH // 16):
            zrows[i, pl.ds(16 * k, 16)] = _zero16()
    zrow1[pl.ds(0, 16)] = _zero16()
    zrow1[pl.ds(PSTR - 16, 16)] = _zero16()

    pltpu.sync_copy(zrows, sum_acc.at[pl.ds(s * PSTR, PSTR)])
    pltpu.sync_copy(zrow1, cnt_acc.at[pl.ds(s * PSTR, PSTR)])

    pltpu.sync_copy(h2d.at[pl.ds(w * RPT, RPT)], h2d_v)
    pltpu.sync_copy(hflat.at[pl.ds(w * RPT * HH, RPT * HH)], hflat_v)
    pltpu.sync_copy(bat2.at[w], bat2_v)
    plsc.subcore_barrier()

    for j in range(NBCH):
        pltpu.sync_copy(h2d_v.at[pl.ds(BCH * j, BCH)],
                        sum_acc.at[bat2_v.at[j]], add=True)
        pltpu.sync_copy(ones_v, cnt_acc.at[bat2_v.at[j]], add=True)

    def mb(i, carry):
        for k in range(HH // 16):
            maxbuf[pl.ds(HH * i + 16 * k, 16)] = ninf16
        return carry

    lax.fori_loop(0, PB, mb, 0)

    carry = (jnp.full((16,), -1, jnp.int32), ninf16, ninf16, ninf16, ninf16)
    for j in range(NBCH):

        def group(g2, c, j=j):
            bprev, c0, c1, c2, c3 = c
            b16 = bat2_v[j, pl.ds(16 * g2, 16)]
            goff = g2 * (16 * HH)
            cur = [c0, c1, c2, c3]
            for r in range(16):
                br = _dyn_gather(b16, jnp.full((16,), r, jnp.int32))
                same = br == bprev
                br64 = br * HH + iota
                rbase = goff + (BCH * j + r) * HH
                for k in range(HH // 16):
                    v = hflat_v[pl.ds(rbase + 16 * k, 16)]
                    cur[k] = jnp.where(same, jnp.maximum(cur[k], v), v)
                    plsc.store_scatter(maxbuf, [br64 + 16 * k], cur[k])
                bprev = br
            return (bprev, cur[0], cur[1], cur[2], cur[3])

        carry = lax.fori_loop(0, BCH // 16, group, carry)

    plsc.subcore_barrier()
    pltpu.sync_copy(sum_acc.at[pl.ds(16 * s, 16)], sstage)
    pltpu.sync_copy(sstage, out_sum.at[pl.ds(c * BB + 16 * s, 16)])
    pltpu.sync_copy(cnt_acc.at[pl.ds(16 * s, 16)], cstage)
    pltpu.sync_copy(cstage, out_cnt.at[pl.ds(c * BB + 16 * s, 16)])
    pltpu.sync_copy(maxbuf.at[pl.ds(0, BB * HH)],
                    out_max.at[pl.ds(w * BB * HH, BB * HH)])


def _pad_t(v):
    return jnp.concatenate(
        [v, jnp.zeros((NACC - NN, HH), jnp.float32)], axis=0)


def _tc_first(x_ref, w_ref, dinv_ref, t_ref):
    h = jnp.dot(x_ref[...], w_ref[...], preferred_element_type=jnp.float32)
    t_ref[...] = _pad_t(h * dinv_ref[...])


def _tc_mid(s_ref, t_ref, dinv_ref, b_ref, w_ref, tn_ref):
    ssum = s_ref[0, :NN, :] + s_ref[1, :NN, :]
    dinv = dinv_ref[...]
    a = jnp.maximum(dinv * (ssum + t_ref[:NN, :]) + b_ref[...], 0.0)
    tn_ref[...] = _pad_t(
        jnp.dot(a, w_ref[...], preferred_element_type=jnp.float32) * dinv)


def _tc_last(s_ref, t_ref, dinv_ref, b_ref, h_ref):
    ssum = s_ref[0, :NN, :] + s_ref[1, :NN, :]
    h_ref[...] = jnp.maximum(
        dinv_ref[...] * (ssum + t_ref[:NN, :]) + b_ref[...], 0.0)


def _tc_out(sum_ref, cnt_ref, max_ref, w_ref, b_ref, out_ref, xp_ref):
    sums = sum_ref[0, :, :] + sum_ref[1, :, :]
    cnt = jnp.sum(cnt_ref[...], axis=0)
    mean = sums / jnp.maximum(cnt, 1.0)[:, None]
    mx = jnp.max(max_ref[...], axis=0)
    xp = jnp.concatenate([mean, mx], axis=1)
    xp_ref[...] = xp
    out_ref[...] = jnp.dot(xp, w_ref[...],
                           preferred_element_type=jnp.float32) + b_ref[...]


def _tc_call(fn, out_shapes):
    return pl.pallas_call(fn, out_shape=out_shapes)


def kernel(x, edge_index, batch_index, W1, b1, W2, b2, W3, b3, W4, b4,
           W_out, b_out):
    f32 = jnp.float32
    src = edge_index[0]
    dst = edge_index[1]
    npad_e = EPAD - EE
    pad_i = jnp.arange(npad_e, dtype=jnp.int32)
    srcs = jnp.concatenate([src, (pad_i * 997) % NN]).reshape(NW, NCH, CHUNK)
    dsts = jnp.concatenate([dst, NN + (pad_i % 16)]).reshape(NW, NCH, CHUNK)

    deg_p = _deg_kernel(dsts)
    deg = deg_p[:NN] + deg_p[NACC:NACC + NN] + 1.0
    dinv = lax.rsqrt(deg)[:, None]

    t1 = _tc_call(_tc_first, jax.ShapeDtypeStruct((NACC, HH), f32))(
        x, W1, dinv)

    t = t1
    for b, W in ((b1, W2), (b2, W3), (b3, W4)):
        s_p = _edge_kernel(t, srcs, dsts).reshape(NC, NACC, HH)
        t = _tc_call(_tc_mid, jax.ShapeDtypeStruct((NACC, HH), f32))(
            s_p, t, dinv, b.reshape(1, HH), W)
    s_p = _edge_kernel(t, srcs, dsts).reshape(NC, NACC, HH)
    h4 = _tc_call(_tc_last, jax.ShapeDtypeStruct((NN, HH), f32))(
        s_p, t, dinv, b4.reshape(1, HH))

    h2d = jnp.pad(h4, ((0, NPAD - NN), (0, 0)))
    hflat = lax.optimization_barrier(h2d).reshape(NPAD * HH)
    pad_b = jnp.arange(NPAD - NN, dtype=jnp.int32)
    batp = jnp.concatenate([batch_index, BB + (pad_b % 16)])
    bat2 = lax.optimization_barrier(batp).reshape(NW, NBCH, BCH)

    sum_p, cnt_p, max_p = _pool_kernel(h2d, hflat, bat2)
    sum_p = sum_p.reshape(NC, BB, HH)
    cnt_p = cnt_p.reshape(NC, BB)
    max_p = max_p.reshape(NW, BB, HH)

    out, xp = _tc_call(_tc_out, (
        jax.ShapeDtypeStruct((BB, 1), f32),
        jax.ShapeDtypeStruct((BB, 2 * HH), f32),
    ))(sum_p, cnt_p, max_p, W_out, b_out.reshape(1, 1))
    return (out, xp)


# speedup vs baseline: 1.2544x; 1.2544x over previous
import functools

import jax
import jax.numpy as jnp
from jax import lax
from jax.experimental import pallas as pl
from jax.experimental.pallas import tpu as pltpu
from jax.experimental.pallas import tpu_sc as plsc

NN = 10000
EE = 320000
DIN = 128
HH = 64
BB = 256

NC = 2
NS = 16
NW = NC * NS

CHUNK = 64
NCH = 160
NBUF = 4
EPW = CHUNK * NCH
EPAD = EPW * NW
NACC = 10112
STRIPE = NACC // NS

NPAD = 10240
RPT = NPAD // NW
NG = RPT // 16
BCH = 64
NBCH = RPT // BCH
PB = 384
PSTR = PB // NS

_MESH = plsc.VectorSubcoreMesh(core_axis_name="c", subcore_axis_name="s")


def _zero16():
    return jnp.zeros((16,), jnp.float32)


def _dyn_gather(x, idx):
    return lax.gather(
        x, idx[:, None],
        lax.GatherDimensionNumbers(offset_dims=(), collapsed_slice_dims=(0,),
                                   start_index_map=(0,)),
        (1,), mode=lax.GatherScatterMode.PROMISE_IN_BOUNDS)


@functools.partial(
    pl.kernel,
    out_type=jax.ShapeDtypeStruct((NC * NACC,), jnp.float32),
    mesh=_MESH,
    scratch_types=[
        pltpu.VMEM((NCH, CHUNK), jnp.int32),
        pltpu.VMEM((CHUNK,), jnp.float32),
        pltpu.VMEM((CHUNK,), jnp.float32),
        pltpu.VMEM((STRIPE,), jnp.float32),
        pltpu.VMEM_SHARED((NACC,), jnp.float32),
    ],
)
def _deg_kernel(dsts, out, dst_v, ones_v, zbuf, stage_v, acc):
    c = lax.axis_index("c")
    s = lax.axis_index("s")
    w = c * NS + s
    one16 = jnp.ones((16,), jnp.float32)
    for k in range(CHUNK // 16):
        ones_v[pl.ds(16 * k, 16)] = one16
        zbuf[pl.ds(16 * k, 16)] = _zero16()
    base = s * STRIPE
    nfull = STRIPE // CHUNK
    tail = STRIPE - nfull * CHUNK
    for q in range(nfull):
        pltpu.sync_copy(zbuf, acc.at[pl.ds(base + q * CHUNK, CHUNK)])
    pltpu.sync_copy(zbuf.at[pl.ds(0, tail)],
                    acc.at[pl.ds(base + nfull * CHUNK, tail)])
    pltpu.sync_copy(dsts.at[w], dst_v)
    plsc.subcore_barrier()

    def step(j, carry):
        pltpu.sync_copy(ones_v, acc.at[dst_v.at[j]], add=True)
        return carry

    lax.fori_loop(0, NCH, step, 0)
    plsc.subcore_barrier()
    pltpu.sync_copy(acc.at[pl.ds(base, STRIPE)], stage_v)
    pltpu.sync_copy(stage_v, out.at[pl.ds(c * NACC + base, STRIPE)])


@functools.partial(
    pl.kernel,
    out_type=jax.ShapeDtypeStruct((NC * NACC, HH), jnp.float32),
    mesh=_MESH,
    compiler_params=pltpu.CompilerParams(use_tc_tiling_on_sc=False),
    scratch_types=[
        pltpu.VMEM((NCH, CHUNK), jnp.int32),
        pltpu.VMEM((NCH, CHUNK), jnp.int32),
    ] + [pltpu.VMEM((CHUNK, HH), jnp.float32)] * NBUF + [
        pltpu.VMEM_SHARED((NACC, HH), jnp.float32),
    ] + [pltpu.SemaphoreType.DMA] * NBUF,
)
def _edge_kernel(t, srcs, dsts, out, src_v, dst_v, r0, r1, r2, r3, acc,
                 m0, m1, m2, m3):
    c = lax.axis_index("c")
    s = lax.axis_index("s")
    w = c * NS + s
    base = s * STRIPE
    bufs = (r0, r1, r2, r3)
    sems = (m0, m1, m2, m3)

    def zb(i, carry):
        for k in range(HH // 16):
            r0[i, pl.ds(16 * k, 16)] = _zero16()
        return carry

    lax.fori_loop(0, CHUNK, zb, 0)
    nfull = STRIPE // CHUNK
    tail = STRIPE - nfull * CHUNK
    for q in range(nfull):
        pltpu.sync_copy(r0, acc.at[pl.ds(base + q * CHUNK, CHUNK)])
    pltpu.sync_copy(r0.at[pl.ds(0, tail)],
                    acc.at[pl.ds(base + nfull * CHUNK, tail)])
    pltpu.sync_copy(srcs.at[w], src_v)
    pltpu.sync_copy(dsts.at[w], dst_v)
    plsc.subcore_barrier()

    for q in range(NBUF):
        pltpu.async_copy(t.at[src_v.at[q]], bufs[q], sems[q])

    def step(g, carry):
        j0 = NBUF * g
        for q in range(NBUF):
            pltpu.make_async_copy(t.at[src_v.at[j0 + q]], bufs[q],
                                  sems[q]).wait()
            pltpu.sync_copy(bufs[q], acc.at[dst_v.at[j0 + q]], add=True)

            @pl.when(j0 + q + NBUF < NCH)
            def _(q=q):
                pltpu.async_copy(t.at[src_v.at[j0 + q + NBUF]], bufs[q],
                                 sems[q])

        return carry

    lax.fori_loop(0, NCH // NBUF, step, 0)
    plsc.subcore_barrier()
    for q in range(nfull):
        pltpu.sync_copy(acc.at[pl.ds(base + q * CHUNK, CHUNK)], r0)
        pltpu.sync_copy(r0,
                        out.at[pl.ds(c * NACC + base + q * CHUNK, CHUNK)])
    pltpu.sync_copy(acc.at[pl.ds(base + nfull * CHUNK, tail)],
                    r0.at[pl.ds(0, tail)])
    pltpu.sync_copy(r0.at[pl.ds(0, tail)],
                    out.at[pl.ds(c * NACC + base + nfull * CHUNK, tail)])


@functools.partial(
    pl.kernel,
    out_type=(
        jax.ShapeDtypeStruct((NC * BB, HH), jnp.float32),
        jax.ShapeDtypeStruct((NC * BB,), jnp.float32),
        jax.ShapeDtypeStruct((NW * BB * HH,), jnp.float32),
    ),
    mesh=_MESH,
    compiler_params=pltpu.CompilerParams(needs_layout_passes=False,
                                         use_tc_tiling_on_sc=False),
    scratch_types=[
        pltpu.VMEM((RPT, HH), jnp.float32),
        pltpu.VMEM((RPT * HH,), jnp.float32),
        pltpu.VMEM((NBCH, BCH), jnp.int32),
        pltpu.VMEM((BCH,), jnp.float32),
        pltpu.VMEM((PSTR, HH), jnp.float32),
        pltpu.VMEM((PSTR,), jnp.float32),
        pltpu.VMEM((PB * HH,), jnp.float32),
        pltpu.VMEM((16, HH), jnp.float32),
        pltpu.VMEM((16,), jnp.float32),
        pltpu.VMEM_SHARED((PB, HH), jnp.float32),
        pltpu.VMEM_SHARED((PB,), jnp.float32),
    ],
)
def _pool_kernel(h2d, hflat, bat2, out_sum, out_cnt, out_max,
                 h2d_v, hflat_v, bat2_v, ones_v, zrows, zrow1,
                 maxbuf, sstage, cstage, sum_acc, cnt_acc):
    c = lax.axis_index("c")
    s = lax.axis_index("s")
    w = c * NS + s
    one16 = jnp.ones((16,), jnp.float32)
    iota = lax.iota(jnp.int32, 16)
    ninf16 = jnp.full((16,), -jnp.inf, jnp.float32)

    for k in range(BCH // 16):
        ones_v[pl.ds(16 * k, 16)] = one16
    for i in range(PSTR):
        for k in range(HH // 16):
            zrows[i, pl.ds(16 * k, 16)] = _zero16()
    zrow1[pl.ds(0, 16)] = _zero16()
    zrow1[pl.ds(PSTR - 16, 16)] = _zero16()

    pltpu.sync_copy(zrows, sum_acc.at[pl.ds(s * PSTR, PSTR)])
    pltpu.sync_copy(zrow1, cnt_acc.at[pl.ds(s * PSTR, PSTR)])

    pltpu.sync_copy(h2d.at[pl.ds(w * RPT, RPT)], h2d_v)
    pltpu.sync_copy(hflat.at[pl.ds(w * RPT * HH, RPT * HH)], hflat_v)
    pltpu.sync_copy(bat2.at[w], bat2_v)
    plsc.subcore_barrier()

    for j in range(NBCH):
        pltpu.sync_copy(h2d_v.at[pl.ds(BCH * j, BCH)],
                        sum_acc.at[bat2_v.at[j]], add=True)
        pltpu.sync_copy(ones_v, cnt_acc.at[bat2_v.at[j]], add=True)

    def mb(i, carry):
        for k in range(HH // 16):
            maxbuf[pl.ds(HH * i + 16 * k, 16)] = ninf16
        return carry

    lax.fori_loop(0, PB, mb, 0)

    carry = (jnp.full((16,), -1, jnp.int32), ninf16, ninf16, ninf16, ninf16)
    for j in range(NBCH):

        def group(g2, c, j=j):
            bprev, c0, c1, c2, c3 = c
            b16 = bat2_v[j, pl.ds(16 * g2, 16)]
            goff = g2 * (16 * HH)
            cur = [c0, c1, c2, c3]
            for r in range(16):
                br = _dyn_gather(b16, jnp.full((16,), r, jnp.int32))
                same = br == bprev
                br64 = br * HH + iota
                rbase = goff + (BCH * j + r) * HH
                for k in range(HH // 16):
                    v = hflat_v[pl.ds(rbase + 16 * k, 16)]
                    cur[k] = jnp.where(same, jnp.maximum(cur[k], v), v)
                    plsc.store_scatter(maxbuf, [br64 + 16 * k], cur[k])
                bprev = br
            return (bprev, cur[0], cur[1], cur[2], cur[3])

        carry = lax.fori_loop(0, BCH // 16, group, carry)

    plsc.subcore_barrier()
    pltpu.sync_copy(sum_acc.at[pl.ds(16 * s, 16)], sstage)
    pltpu.sync_copy(sstage, out_sum.at[pl.ds(c * BB + 16 * s, 16)])
    pltpu.sync_copy(cnt_acc.at[pl.ds(16 * s, 16)], cstage)
    pltpu.sync_copy(cstage, out_cnt.at[pl.ds(c * BB + 16 * s, 16)])
    pltpu.sync_copy(maxbuf.at[pl.ds(0, BB * HH)],
                    out_max.at[pl.ds(w * BB * HH, BB * HH)])


def _pad_t(v):
    return jnp.concatenate(
        [v, jnp.zeros((NACC - NN, HH), jnp.float32)], axis=0)


def _tc_first(x_ref, w_ref, dinv_ref, t_ref):
    h = jnp.dot(x_ref[...], w_ref[...], preferred_element_type=jnp.float32)
    t_ref[...] = _pad_t(h * dinv_ref[...])


def _tc_mid(s_ref, t_ref, dinv_ref, b_ref, w_ref, tn_ref):
    ssum = s_ref[0, :NN, :] + s_ref[1, :NN, :]
    dinv = dinv_ref[...]
    a = jnp.maximum(dinv * (ssum + t_ref[:NN, :]) + b_ref[...], 0.0)
    tn_ref[...] = _pad_t(
        jnp.dot(a, w_ref[...], preferred_element_type=jnp.float32) * dinv)


def _tc_last(s_ref, t_ref, dinv_ref, b_ref, h_ref):
    ssum = s_ref[0, :NN, :] + s_ref[1, :NN, :]
    h_ref[...] = jnp.maximum(
        dinv_ref[...] * (ssum + t_ref[:NN, :]) + b_ref[...], 0.0)


def _tc_out(sum_ref, cnt_ref, max_ref, w_ref, b_ref, out_ref, xp_ref):
    sums = sum_ref[0, :, :] + sum_ref[1, :, :]
    cnt = jnp.sum(cnt_ref[...], axis=0)
    mean = sums / jnp.maximum(cnt, 1.0)[:, None]
    mx = jnp.max(max_ref[...], axis=0)
    xp = jnp.concatenate([mean, mx], axis=1)
    xp_ref[...] = xp
    out_ref[...] = jnp.dot(xp, w_ref[...],
                           preferred_element_type=jnp.float32) + b_ref[...]


def _tc_call(fn, out_shapes):
    return pl.pallas_call(fn, out_shape=out_shapes)


def kernel(x, edge_index, batch_index, W1, b1, W2, b2, W3, b3, W4, b4,
           W_out, b_out):
    f32 = jnp.float32
    src = edge_index[0]
    dst = edge_index[1]
    npad_e = EPAD - EE
    pad_i = jnp.arange(npad_e, dtype=jnp.int32)
    srcs = jnp.concatenate([src, (pad_i * 997) % NN]).reshape(NW, NCH, CHUNK)
    dsts = jnp.concatenate([dst, NN + (pad_i % 16)]).reshape(NW, NCH, CHUNK)

    deg_p = _deg_kernel(dsts)
    deg = deg_p[:NN] + deg_p[NACC:NACC + NN] + 1.0
    dinv = lax.rsqrt(deg)[:, None]

    t1 = _tc_call(_tc_first, jax.ShapeDtypeStruct((NACC, HH), f32))(
        x, W1, dinv)

    t = t1
    for b, W in ((b1, W2), (b2, W3), (b3, W4)):
        s_p = _edge_kernel(t, srcs, dsts).reshape(NC, NACC, HH)
        t = _tc_call(_tc_mid, jax.ShapeDtypeStruct((NACC, HH), f32))(
            s_p, t, dinv, b.reshape(1, HH), W)
    s_p = _edge_kernel(t, srcs, dsts).reshape(NC, NACC, HH)
    h4 = _tc_call(_tc_last, jax.ShapeDtypeStruct((NN, HH), f32))(
        s_p, t, dinv, b4.reshape(1, HH))

    h2d = jnp.pad(h4, ((0, NPAD - NN), (0, 0)))
    hflat = lax.optimization_barrier(h2d).reshape(NPAD * HH)
    pad_b = jnp.arange(NPAD - NN, dtype=jnp.int32)
    batp = jnp.concatenate([batch_index, BB + (pad_b % 16)])
    bat2 = lax.optimization_barrier(batp).reshape(NW, NBCH, BCH)

    sum_p, cnt_p, max_p = _pool_kernel(h2d, hflat, bat2)
    sum_p = sum_p.reshape(NC, BB, HH)
    cnt_p = cnt_p.reshape(NC, BB)
    max_p = max_p.reshape(NW, BB, HH)

    out, xp = _tc_call(_tc_out, (
        jax.ShapeDtypeStruct((BB, 1), f32),
        jax.ShapeDtypeStruct((BB, 2 * HH), f32),
    ))(sum_p, cnt_p, max_p, W_out, b_out.reshape(1, 1))
    return (out, xp)


# speedup vs baseline: 1.3491x; 1.0755x over previous
import functools

import jax
import jax.numpy as jnp
from jax import lax
from jax.experimental import pallas as pl
from jax.experimental.pallas import tpu as pltpu
from jax.experimental.pallas import tpu_sc as plsc

NN = 10000
EE = 320000
DIN = 128
HH = 64
BB = 256

NC = 2
NS = 16
NW = NC * NS

CHUNK = 128
NCH = 80
NBUF = 8
EPW = CHUNK * NCH
EPAD = EPW * NW
NACC = 10112
STRIPE = NACC // NS

NPAD = 10240
RPT = NPAD // NW
NG = RPT // 16
BCH = 64
NBCH = RPT // BCH
PB = 384
PSTR = PB // NS

_MESH = plsc.VectorSubcoreMesh(core_axis_name="c", subcore_axis_name="s")


def _zero16():
    return jnp.zeros((16,), jnp.float32)


def _dyn_gather(x, idx):
    return lax.gather(
        x, idx[:, None],
        lax.GatherDimensionNumbers(offset_dims=(), collapsed_slice_dims=(0,),
                                   start_index_map=(0,)),
        (1,), mode=lax.GatherScatterMode.PROMISE_IN_BOUNDS)


@functools.partial(
    pl.kernel,
    out_type=jax.ShapeDtypeStruct((NC * NACC,), jnp.float32),
    mesh=_MESH,
    scratch_types=[
        pltpu.VMEM((NCH, CHUNK), jnp.int32),
        pltpu.VMEM((CHUNK,), jnp.float32),
        pltpu.VMEM((CHUNK,), jnp.float32),
        pltpu.VMEM((STRIPE,), jnp.float32),
        pltpu.VMEM_SHARED((NACC,), jnp.float32),
    ],
)
def _deg_kernel(dsts, out, dst_v, ones_v, zbuf, stage_v, acc):
    c = lax.axis_index("c")
    s = lax.axis_index("s")
    w = c * NS + s
    one16 = jnp.ones((16,), jnp.float32)
    for k in range(CHUNK // 16):
        ones_v[pl.ds(16 * k, 16)] = one16
        zbuf[pl.ds(16 * k, 16)] = _zero16()
    base = s * STRIPE
    nfull = STRIPE // CHUNK
    tail = STRIPE - nfull * CHUNK
    for q in range(nfull):
        pltpu.sync_copy(zbuf, acc.at[pl.ds(base + q * CHUNK, CHUNK)])
    pltpu.sync_copy(zbuf.at[pl.ds(0, tail)],
                    acc.at[pl.ds(base + nfull * CHUNK, tail)])
    pltpu.sync_copy(dsts.at[w], dst_v)
    plsc.subcore_barrier()

    def step(j, carry):
        pltpu.sync_copy(ones_v, acc.at[dst_v.at[j]], add=True)
        return carry

    lax.fori_loop(0, NCH, step, 0)
    plsc.subcore_barrier()
    pltpu.sync_copy(acc.at[pl.ds(base, STRIPE)], stage_v)
    pltpu.sync_copy(stage_v, out.at[pl.ds(c * NACC + base, STRIPE)])


@functools.partial(
    pl.kernel,
    out_type=jax.ShapeDtypeStruct((NC * NACC, HH), jnp.float32),
    mesh=_MESH,
    compiler_params=pltpu.CompilerParams(use_tc_tiling_on_sc=False),
    scratch_types=[
        pltpu.VMEM((NCH, CHUNK), jnp.int32),
        pltpu.VMEM((NCH, CHUNK), jnp.int32),
    ] + [pltpu.VMEM((CHUNK, HH), jnp.float32)] * NBUF + [
        pltpu.VMEM_SHARED((NACC, HH), jnp.float32),
    ] + [pltpu.SemaphoreType.DMA] * NBUF,
)
def _edge_kernel(t, srcs, dsts, out, src_v, dst_v, r0, r1, r2, r3, r4, r5,
                 r6, r7, acc, m0, m1, m2, m3, m4, m5, m6, m7):
    c = lax.axis_index("c")
    s = lax.axis_index("s")
    w = c * NS + s
    base = s * STRIPE
    bufs = (r0, r1, r2, r3, r4, r5, r6, r7)
    sems = (m0, m1, m2, m3, m4, m5, m6, m7)

    def zb(i, carry):
        for k in range(HH // 16):
            r0[i, pl.ds(16 * k, 16)] = _zero16()
        return carry

    lax.fori_loop(0, CHUNK, zb, 0)
    nfull = STRIPE // CHUNK
    tail = STRIPE - nfull * CHUNK
    for q in range(nfull):
        pltpu.sync_copy(r0, acc.at[pl.ds(base + q * CHUNK, CHUNK)])
    pltpu.sync_copy(r0.at[pl.ds(0, tail)],
                    acc.at[pl.ds(base + nfull * CHUNK, tail)])
    pltpu.sync_copy(srcs.at[w], src_v)
    pltpu.sync_copy(dsts.at[w], dst_v)
    plsc.subcore_barrier()

    for q in range(NBUF):
        pltpu.async_copy(t.at[src_v.at[q]], bufs[q], sems[q])

    def step(g, carry):
        j0 = NBUF * g
        for q in range(NBUF):
            pltpu.make_async_copy(t.at[src_v.at[j0 + q]], bufs[q],
                                  sems[q]).wait()
            pltpu.sync_copy(bufs[q], acc.at[dst_v.at[j0 + q]], add=True)
            pltpu.async_copy(t.at[src_v.at[j0 + q + NBUF]], bufs[q],
                             sems[q])
        return carry

    lax.fori_loop(0, NCH // NBUF - 1, step, 0)
    jlast = NCH - NBUF
    for q in range(NBUF):
        pltpu.make_async_copy(t.at[src_v.at[jlast + q]], bufs[q],
                              sems[q]).wait()
        pltpu.sync_copy(bufs[q], acc.at[dst_v.at[jlast + q]], add=True)
    plsc.subcore_barrier()
    for q in range(nfull):
        pltpu.sync_copy(acc.at[pl.ds(base + q * CHUNK, CHUNK)], r0)
        pltpu.sync_copy(r0,
                        out.at[pl.ds(c * NACC + base + q * CHUNK, CHUNK)])
    pltpu.sync_copy(acc.at[pl.ds(base + nfull * CHUNK, tail)],
                    r0.at[pl.ds(0, tail)])
    pltpu.sync_copy(r0.at[pl.ds(0, tail)],
                    out.at[pl.ds(c * NACC + base + nfull * CHUNK, tail)])


@functools.partial(
    pl.kernel,
    out_type=(
        jax.ShapeDtypeStruct((NC * BB, HH), jnp.float32),
        jax.ShapeDtypeStruct((NC * BB,), jnp.float32),
        jax.ShapeDtypeStruct((NW * BB * HH,), jnp.float32),
    ),
    mesh=_MESH,
    compiler_params=pltpu.CompilerParams(needs_layout_passes=False,
                                         use_tc_tiling_on_sc=False),
    scratch_types=[
        pltpu.VMEM((RPT, HH), jnp.float32),
        pltpu.VMEM((RPT * HH,), jnp.float32),
        pltpu.VMEM((NBCH, BCH), jnp.int32),
        pltpu.VMEM((BCH,), jnp.float32),
        pltpu.VMEM((PSTR, HH), jnp.float32),
        pltpu.VMEM((PSTR,), jnp.float32),
        pltpu.VMEM((PB * HH,), jnp.float32),
        pltpu.VMEM((16, HH), jnp.float32),
        pltpu.VMEM((16,), jnp.float32),
        pltpu.VMEM_SHARED((PB, HH), jnp.float32),
        pltpu.VMEM_SHARED((PB,), jnp.float32),
    ],
)
def _pool_kernel(h2d, hflat, bat2, out_sum, out_cnt, out_max,
                 h2d_v, hflat_v, bat2_v, ones_v, zrows, zrow1,
                 maxbuf, sstage, cstage, sum_acc, cnt_acc):
    c = lax.axis_index("c")
    s = lax.axis_index("s")
    w = c * NS + s
    one16 = jnp.ones((16,), jnp.float32)
    iota = lax.iota(jnp.int32, 16)
    ninf16 = jnp.full((16,), -jnp.inf, jnp.float32)

    for k in range(BCH // 16):
        ones_v[pl.ds(16 * k, 16)] = one16
    for i in range(PSTR):
        for k in range(HH // 16):
            zrows[i, pl.ds(16 * k, 16)] = _zero16()
    zrow1[pl.ds(0, 16)] = _zero16()
    zrow1[pl.ds(PSTR - 16, 16)] = _zero16()

    pltpu.sync_copy(zrows, sum_acc.at[pl.ds(s * PSTR, PSTR)])
    pltpu.sync_copy(zrow1, cnt_acc.at[pl.ds(s * PSTR, PSTR)])

    pltpu.sync_copy(h2d.at[pl.ds(w * RPT, RPT)], h2d_v)
    pltpu.sync_copy(hflat.at[pl.ds(w * RPT * HH, RPT * HH)], hflat_v)
    pltpu.sync_copy(bat2.at[w], bat2_v)
    plsc.subcore_barrier()

    for j in range(NBCH):
        pltpu.sync_copy(h2d_v.at[pl.ds(BCH * j, BCH)],
                        sum_acc.at[bat2_v.at[j]], add=True)
        pltpu.sync_copy(ones_v, cnt_acc.at[bat2_v.at[j]], add=True)

    def mb(i, carry):
        for k in range(HH // 16):
            maxbuf[pl.ds(HH * i + 16 * k, 16)] = ninf16
        return carry

    lax.fori_loop(0, PB, mb, 0)

    carry = (jnp.full((16,), -1, jnp.int32), ninf16, ninf16, ninf16, ninf16)
    for j in range(NBCH):

        def group(g2, c, j=j):
            bprev, c0, c1, c2, c3 = c
            b16 = bat2_v[j, pl.ds(16 * g2, 16)]
            goff = g2 * (16 * HH)
            cur = [c0, c1, c2, c3]
            for r in range(16):
                br = _dyn_gather(b16, jnp.full((16,), r, jnp.int32))
                same = br == bprev
                br64 = br * HH + iota
                rbase = goff + (BCH * j + r) * HH
                for k in range(HH // 16):
                    v = hflat_v[pl.ds(rbase + 16 * k, 16)]
                    cur[k] = jnp.where(same, jnp.maximum(cur[k], v), v)
                    plsc.store_scatter(maxbuf, [br64 + 16 * k], cur[k])
                bprev = br
            return (bprev, cur[0], cur[1], cur[2], cur[3])

        carry = lax.fori_loop(0, BCH // 16, group, carry)

    plsc.subcore_barrier()
    pltpu.sync_copy(sum_acc.at[pl.ds(16 * s, 16)], sstage)
    pltpu.sync_copy(sstage, out_sum.at[pl.ds(c * BB + 16 * s, 16)])
    pltpu.sync_copy(cnt_acc.at[pl.ds(16 * s, 16)], cstage)
    pltpu.sync_copy(cstage, out_cnt.at[pl.ds(c * BB + 16 * s, 16)])
    pltpu.sync_copy(maxbuf.at[pl.ds(0, BB * HH)],
                    out_max.at[pl.ds(w * BB * HH, BB * HH)])


def _pad_t(v):
    return jnp.concatenate(
        [v, jnp.zeros((NACC - NN, HH), jnp.float32)], axis=0)


def _tc_first(x_ref, w_ref, dinv_ref, t_ref):
    h = jnp.dot(x_ref[...], w_ref[...], preferred_element_type=jnp.float32)
    t_ref[...] = _pad_t(h * dinv_ref[...])


def _tc_mid(s_ref, t_ref, dinv_ref, b_ref, w_ref, tn_ref):
    ssum = s_ref[0, :NN, :] + s_ref[1, :NN, :]
    dinv = dinv_ref[...]
    a = jnp.maximum(dinv * (ssum + t_ref[:NN, :]) + b_ref[...], 0.0)
    tn_ref[...] = _pad_t(
        jnp.dot(a, w_ref[...], preferred_element_type=jnp.float32) * dinv)


def _tc_last(s_ref, t_ref, dinv_ref, b_ref, h_ref):
    ssum = s_ref[0, :NN, :] + s_ref[1, :NN, :]
    h_ref[...] = jnp.maximum(
        dinv_ref[...] * (ssum + t_ref[:NN, :]) + b_ref[...], 0.0)


def _tc_out(sum_ref, cnt_ref, max_ref, w_ref, b_ref, out_ref, xp_ref):
    sums = sum_ref[0, :, :] + sum_ref[1, :, :]
    cnt = jnp.sum(cnt_ref[...], axis=0)
    mean = sums / jnp.maximum(cnt, 1.0)[:, None]
    mx = jnp.max(max_ref[...], axis=0)
    xp = jnp.concatenate([mean, mx], axis=1)
    xp_ref[...] = xp
    out_ref[...] = jnp.dot(xp, w_ref[...],
                           preferred_element_type=jnp.float32) + b_ref[...]


def _tc_call(fn, out_shapes):
    return pl.pallas_call(fn, out_shape=out_shapes)


def kernel(x, edge_index, batch_index, W1, b1, W2, b2, W3, b3, W4, b4,
           W_out, b_out):
    f32 = jnp.float32
    src = edge_index[0]
    dst = edge_index[1]
    npad_e = EPAD - EE
    pad_i = jnp.arange(npad_e, dtype=jnp.int32)
    srcs = jnp.concatenate([src, (pad_i * 997) % NN]).reshape(NW, NCH, CHUNK)
    dsts = jnp.concatenate([dst, NN + (pad_i % 16)]).reshape(NW, NCH, CHUNK)

    deg_p = _deg_kernel(dsts)
    deg = deg_p[:NN] + deg_p[NACC:NACC + NN] + 1.0
    dinv = lax.rsqrt(deg)[:, None]

    t1 = _tc_call(_tc_first, jax.ShapeDtypeStruct((NACC, HH), f32))(
        x, W1, dinv)

    t = t1
    for b, W in ((b1, W2), (b2, W3), (b3, W4)):
        s_p = _edge_kernel(t, srcs, dsts).reshape(NC, NACC, HH)
        t = _tc_call(_tc_mid, jax.ShapeDtypeStruct((NACC, HH), f32))(
            s_p, t, dinv, b.reshape(1, HH), W)
    s_p = _edge_kernel(t, srcs, dsts).reshape(NC, NACC, HH)
    h4 = _tc_call(_tc_last, jax.ShapeDtypeStruct((NN, HH), f32))(
        s_p, t, dinv, b4.reshape(1, HH))

    h2d = jnp.pad(h4, ((0, NPAD - NN), (0, 0)))
    hflat = lax.optimization_barrier(h2d).reshape(NPAD * HH)
    pad_b = jnp.arange(NPAD - NN, dtype=jnp.int32)
    batp = jnp.concatenate([batch_index, BB + (pad_b % 16)])
    bat2 = lax.optimization_barrier(batp).reshape(NW, NBCH, BCH)

    sum_p, cnt_p, max_p = _pool_kernel(h2d, hflat, bat2)
    sum_p = sum_p.reshape(NC, BB, HH)
    cnt_p = cnt_p.reshape(NC, BB)
    max_p = max_p.reshape(NW, BB, HH)

    out, xp = _tc_call(_tc_out, (
        jax.ShapeDtypeStruct((BB, 1), f32),
        jax.ShapeDtypeStruct((BB, 2 * HH), f32),
    ))(sum_p, cnt_p, max_p, W_out, b_out.reshape(1, 1))
    return (out, xp)


# speedup vs baseline: 1.3504x; 1.0010x over previous
import functools

import jax
import jax.numpy as jnp
from jax import lax
from jax.experimental import pallas as pl
from jax.experimental.pallas import tpu as pltpu
from jax.experimental.pallas import tpu_sc as plsc

NN = 10000
EE = 320000
DIN = 128
HH = 64
BB = 256

NC = 2
NS = 16
NW = NC * NS

CHUNK = 128
NCH = 80
NBUF = 8
EPW = CHUNK * NCH
EPAD = EPW * NW
NACC = 10112
STRIPE = NACC // NS

NPAD = 10240
RPT = NPAD // NW
BCH = 64
NBCH = RPT // BCH
PB = 384
PSTR = PB // NS

_MESH = plsc.VectorSubcoreMesh(core_axis_name="c", subcore_axis_name="s")


def _zero16():
    return jnp.zeros((16,), jnp.float32)


def _dyn_gather(x, idx):
    return lax.gather(
        x, idx[:, None],
        lax.GatherDimensionNumbers(offset_dims=(), collapsed_slice_dims=(0,),
                                   start_index_map=(0,)),
        (1,), mode=lax.GatherScatterMode.PROMISE_IN_BOUNDS)


@functools.partial(
    pl.kernel,
    out_type=jax.ShapeDtypeStruct((NC * NACC,), jnp.float32),
    mesh=_MESH,
    scratch_types=[
        pltpu.VMEM((NCH, CHUNK), jnp.int32),
        pltpu.VMEM((CHUNK,), jnp.float32),
        pltpu.VMEM((CHUNK,), jnp.float32),
        pltpu.VMEM((STRIPE,), jnp.float32),
        pltpu.VMEM_SHARED((NACC,), jnp.float32),
    ],
)
def _deg_kernel(dsts, out, dst_v, ones_v, zbuf, stage_v, acc):
    c = lax.axis_index("c")
    s = lax.axis_index("s")
    w = c * NS + s
    one16 = jnp.ones((16,), jnp.float32)
    for k in range(CHUNK // 16):
        ones_v[pl.ds(16 * k, 16)] = one16
        zbuf[pl.ds(16 * k, 16)] = _zero16()
    base = s * STRIPE
    nfull = STRIPE // CHUNK
    tail = STRIPE - nfull * CHUNK
    for q in range(nfull):
        pltpu.sync_copy(zbuf, acc.at[pl.ds(base + q * CHUNK, CHUNK)])
    pltpu.sync_copy(zbuf.at[pl.ds(0, tail)],
                    acc.at[pl.ds(base + nfull * CHUNK, tail)])
    pltpu.sync_copy(dsts.at[w], dst_v)
    plsc.subcore_barrier()

    def step(j, carry):
        pltpu.sync_copy(ones_v, acc.at[dst_v.at[j]], add=True)
        return carry

    lax.fori_loop(0, NCH, step, 0)
    plsc.subcore_barrier()
    pltpu.sync_copy(acc.at[pl.ds(base, STRIPE)], stage_v)
    pltpu.sync_copy(stage_v, out.at[pl.ds(c * NACC + base, STRIPE)])


@functools.partial(
    pl.kernel,
    out_type=jax.ShapeDtypeStruct((NC * NACC, HH), jnp.float32),
    mesh=_MESH,
    compiler_params=pltpu.CompilerParams(use_tc_tiling_on_sc=False),
    scratch_types=[
        pltpu.VMEM((NCH, CHUNK), jnp.int32),
        pltpu.VMEM((NCH, CHUNK), jnp.int32),
    ] + [pltpu.VMEM((CHUNK, HH), jnp.float32)] * NBUF + [
        pltpu.VMEM_SHARED((NACC, HH), jnp.float32),
    ] + [pltpu.SemaphoreType.DMA] * NBUF,
)
def _edge_kernel(t, srcs, dsts, out, src_v, dst_v, r0, r1, r2, r3, r4, r5,
                 r6, r7, acc, m0, m1, m2, m3, m4, m5, m6, m7):
    c = lax.axis_index("c")
    s = lax.axis_index("s")
    w = c * NS + s
    base = s * STRIPE
    bufs = (r0, r1, r2, r3, r4, r5, r6, r7)
    sems = (m0, m1, m2, m3, m4, m5, m6, m7)

    pltpu.sync_copy(srcs.at[w], src_v)
    for q in range(NBUF - 1):
        pltpu.async_copy(t.at[src_v.at[q]], bufs[q], sems[q])
    pltpu.sync_copy(dsts.at[w], dst_v)
    zlast = bufs[NBUF - 1]

    def zb(i, carry):
        for k in range(HH // 16):
            zlast[i, pl.ds(16 * k, 16)] = _zero16()
        return carry

    lax.fori_loop(0, CHUNK, zb, 0)
    nfull = STRIPE // CHUNK
    tail = STRIPE - nfull * CHUNK
    for q in range(nfull):
        pltpu.sync_copy(zlast, acc.at[pl.ds(base + q * CHUNK, CHUNK)])
    pltpu.sync_copy(zlast.at[pl.ds(0, tail)],
                    acc.at[pl.ds(base + nfull * CHUNK, tail)])
    pltpu.async_copy(t.at[src_v.at[NBUF - 1]], zlast, sems[NBUF - 1])
    plsc.subcore_barrier()

    def step(g, carry):
        j0 = NBUF * g
        for q in range(NBUF):
            pltpu.make_async_copy(t.at[src_v.at[j0 + q]], bufs[q],
                                  sems[q]).wait()
            pltpu.sync_copy(bufs[q], acc.at[dst_v.at[j0 + q]], add=True)
            pltpu.async_copy(t.at[src_v.at[j0 + q + NBUF]], bufs[q],
                             sems[q])
        return carry

    lax.fori_loop(0, NCH // NBUF - 1, step, 0)
    jlast = NCH - NBUF
    for q in range(NBUF):
        pltpu.make_async_copy(t.at[src_v.at[jlast + q]], bufs[q],
                              sems[q]).wait()
        pltpu.sync_copy(bufs[q], acc.at[dst_v.at[jlast + q]], add=True)
    plsc.subcore_barrier()
    for q in range(nfull):
        pltpu.sync_copy(acc.at[pl.ds(base + q * CHUNK, CHUNK)], r0)
        pltpu.sync_copy(r0,
                        out.at[pl.ds(c * NACC + base + q * CHUNK, CHUNK)])
    pltpu.sync_copy(acc.at[pl.ds(base + nfull * CHUNK, tail)],
                    r0.at[pl.ds(0, tail)])
    pltpu.sync_copy(r0.at[pl.ds(0, tail)],
                    out.at[pl.ds(c * NACC + base + nfull * CHUNK, tail)])


@functools.partial(
    pl.kernel,
    out_type=(
        jax.ShapeDtypeStruct((NC * BB, HH), jnp.float32),
        jax.ShapeDtypeStruct((NC * BB,), jnp.float32),
        jax.ShapeDtypeStruct((NW * BB * HH,), jnp.float32),
    ),
    mesh=_MESH,
    compiler_params=pltpu.CompilerParams(needs_layout_passes=False,
                                         use_tc_tiling_on_sc=False),
    scratch_types=[
        pltpu.VMEM((RPT, HH), jnp.float32),
        pltpu.VMEM((RPT * HH,), jnp.float32),
        pltpu.VMEM((NBCH, BCH), jnp.int32),
        pltpu.VMEM((BCH,), jnp.float32),
        pltpu.VMEM((PSTR, HH), jnp.float32),
        pltpu.VMEM((PSTR,), jnp.float32),
        pltpu.VMEM((PB * HH,), jnp.float32),
        pltpu.VMEM((16, HH), jnp.float32),
        pltpu.VMEM((16,), jnp.float32),
        pltpu.VMEM_SHARED((PB, HH), jnp.float32),
        pltpu.VMEM_SHARED((PB,), jnp.float32),
    ],
)
def _pool_kernel(h2d, hflat, bat2, out_sum, out_cnt, out_max,
                 h2d_v, hflat_v, bat2_v, ones_v, zrows, zrow1,
                 maxbuf, sstage, cstage, sum_acc, cnt_acc):
    c = lax.axis_index("c")
    s = lax.axis_index("s")
    w = c * NS + s
    one16 = jnp.ones((16,), jnp.float32)
    iota = lax.iota(jnp.int32, 16)
    ninf16 = jnp.full((16,), -jnp.inf, jnp.float32)

    for k in range(BCH // 16):
        ones_v[pl.ds(16 * k, 16)] = one16
    for i in range(PSTR):
        for k in range(HH // 16):
            zrows[i, pl.ds(16 * k, 16)] = _zero16()
    zrow1[pl.ds(0, 16)] = _zero16()
    zrow1[pl.ds(PSTR - 16, 16)] = _zero16()

    pltpu.sync_copy(zrows, sum_acc.at[pl.ds(s * PSTR, PSTR)])
    pltpu.sync_copy(zrow1, cnt_acc.at[pl.ds(s * PSTR, PSTR)])

    pltpu.sync_copy(h2d.at[pl.ds(w * RPT, RPT)], h2d_v)
    pltpu.sync_copy(hflat.at[pl.ds(w * RPT * HH, RPT * HH)], hflat_v)
    pltpu.sync_copy(bat2.at[w], bat2_v)
    plsc.subcore_barrier()

    for j in range(NBCH):
        pltpu.sync_copy(h2d_v.at[pl.ds(BCH * j, BCH)],
                        sum_acc.at[bat2_v.at[j]], add=True)
        pltpu.sync_copy(ones_v, cnt_acc.at[bat2_v.at[j]], add=True)

    def mb(i, carry):
        for k in range(HH // 16):
            maxbuf[pl.ds(HH * i + 16 * k, 16)] = ninf16
        return carry

    lax.fori_loop(0, PB, mb, 0)

    carry = (jnp.full((16,), -1, jnp.int32), ninf16, ninf16, ninf16, ninf16)
    for j in range(NBCH):

        def group(g2, c, j=j):
            bprev, c0, c1, c2, c3 = c
            b16 = bat2_v[j, pl.ds(16 * g2, 16)]
            goff = g2 * (16 * HH)
            cur = [c0, c1, c2, c3]
            for r in range(16):
                br = _dyn_gather(b16, jnp.full((16,), r, jnp.int32))
                same = br == bprev
                br64 = br * HH + iota
                rbase = goff + (BCH * j + r) * HH
                for k in range(HH // 16):
                    v = hflat_v[pl.ds(rbase + 16 * k, 16)]
                    cur[k] = jnp.where(same, jnp.maximum(cur[k], v), v)
                    plsc.store_scatter(maxbuf, [br64 + 16 * k], cur[k])
                bprev = br
            return (bprev, cur[0], cur[1], cur[2], cur[3])

        carry = lax.fori_loop(0, BCH // 16, group, carry)

    plsc.subcore_barrier()
    pltpu.sync_copy(sum_acc.at[pl.ds(16 * s, 16)], sstage)
    pltpu.sync_copy(sstage, out_sum.at[pl.ds(c * BB + 16 * s, 16)])
    pltpu.sync_copy(cnt_acc.at[pl.ds(16 * s, 16)], cstage)
    pltpu.sync_copy(cstage, out_cnt.at[pl.ds(c * BB + 16 * s, 16)])
    pltpu.sync_copy(maxbuf.at[pl.ds(0, BB * HH)],
                    out_max.at[pl.ds(w * BB * HH, BB * HH)])


def _pad_t(v):
    return jnp.concatenate(
        [v, jnp.zeros((NACC - NN, HH), jnp.float32)], axis=0)


def _tc_mm(x_ref, w_ref, u_ref):
    u_ref[...] = jnp.dot(x_ref[...], w_ref[...],
                         preferred_element_type=jnp.float32)


def _tc_scale(u_ref, dinv_ref, t_ref):
    t_ref[...] = _pad_t(u_ref[...] * dinv_ref[...])


def _tc_mid(s_ref, t_ref, dinv_ref, b_ref, w_ref, tn_ref):
    ssum = s_ref[0, :NN, :] + s_ref[1, :NN, :]
    dinv = dinv_ref[...]
    a = jnp.maximum(dinv * (ssum + t_ref[:NN, :]) + b_ref[...], 0.0)
    tn_ref[...] = _pad_t(
        jnp.dot(a, w_ref[...], preferred_element_type=jnp.float32) * dinv)


def _tc_last(s_ref, t_ref, dinv_ref, b_ref, h_ref):
    ssum = s_ref[0, :NN, :] + s_ref[1, :NN, :]
    h_ref[...] = jnp.maximum(
        dinv_ref[...] * (ssum + t_ref[:NN, :]) + b_ref[...], 0.0)


def _tc_out(sum_ref, cnt_ref, max_ref, w_ref, b_ref, out_ref, xp_ref):
    sums = sum_ref[0, :, :] + sum_ref[1, :, :]
    cnt = jnp.sum(cnt_ref[...], axis=0)
    mean = sums / jnp.maximum(cnt, 1.0)[:, None]
    mx = jnp.max(max_ref[...], axis=0)
    xp = jnp.concatenate([mean, mx], axis=1)
    xp_ref[...] = xp
    out_ref[...] = jnp.dot(xp, w_ref[...],
                           preferred_element_type=jnp.float32) + b_ref[...]


def _tc_call(fn, out_shapes):
    return pl.pallas_call(fn, out_shape=out_shapes)


def kernel(x, edge_index, batch_index, W1, b1, W2, b2, W3, b3, W4, b4,
           W_out, b_out):
    f32 = jnp.float32
    src = edge_index[0]
    dst = edge_index[1]
    npad_e = EPAD - EE
    pad_i = jnp.arange(npad_e, dtype=jnp.int32)
    srcs = jnp.concatenate([src, (pad_i * 997) % NN]).reshape(NW, NCH, CHUNK)
    dsts = jnp.concatenate([dst, NN + (pad_i % 16)]).reshape(NW, NCH, CHUNK)

    deg_p = _deg_kernel(dsts)
    u1 = _tc_call(_tc_mm, jax.ShapeDtypeStruct((NN, HH), f32))(x, W1)
    deg = deg_p[:NN] + deg_p[NACC:NACC + NN] + 1.0
    dinv = lax.rsqrt(deg)[:, None]
    t1 = _tc_call(_tc_scale, jax.ShapeDtypeStruct((NACC, HH), f32))(u1, dinv)

    t = t1
    for b, W in ((b1, W2), (b2, W3), (b3, W4)):
        s_p = _edge_kernel(t, srcs, dsts).reshape(NC, NACC, HH)
        t = _tc_call(_tc_mid, jax.ShapeDtypeStruct((NACC, HH), f32))(
            s_p, t, dinv, b.reshape(1, HH), W)
    s_p = _edge_kernel(t, srcs, dsts).reshape(NC, NACC, HH)
    h4 = _tc_call(_tc_last, jax.ShapeDtypeStruct((NN, HH), f32))(
        s_p, t, dinv, b4.reshape(1, HH))

    h2d = jnp.pad(h4, ((0, NPAD - NN), (0, 0)))
    hflat = lax.optimization_barrier(h2d).reshape(NPAD * HH)
    pad_b = jnp.arange(NPAD - NN, dtype=jnp.int32)
    batp = jnp.concatenate([batch_index, BB + (pad_b % 16)])
    bat2 = lax.optimization_barrier(batp).reshape(NW, NBCH, BCH)

    sum_p, cnt_p, max_p = _pool_kernel(h2d, hflat, bat2)
    sum_p = sum_p.reshape(NC, BB, HH)
    cnt_p = cnt_p.reshape(NC, BB)
    max_p = max_p.reshape(NW, BB, HH)

    out, xp = _tc_call(_tc_out, (
        jax.ShapeDtypeStruct((BB, 1), f32),
        jax.ShapeDtypeStruct((BB, 2 * HH), f32),
    ))(sum_p, cnt_p, max_p, W_out, b_out.reshape(1, 1))
    return (out, xp)
